# baseline (speedup 1.0000x reference)
import jax
import jax.numpy as jnp
from jax import lax
from jax.experimental import pallas as pl

HEAD_DIM = 64
NUM_EMB = 257
K_PAD = 264
M_BLOCK = 8192
CHUNK = 2048


def _gather_kernel(idx_ref, emb_hi_ref, emb_lo_ref, out_ref):
    emb_hi = emb_hi_ref[...]
    emb_lo = emb_lo_ref[...]
    for c in range(M_BLOCK // CHUNK):
        sl = pl.ds(c * CHUNK, CHUNK)
        idx = idx_ref[sl, :]
        iota = lax.broadcasted_iota(jnp.int32, (CHUNK, K_PAD), 1)
        onehot = jnp.where(idx == iota, 1.0, 0.0).astype(jnp.bfloat16)
        hi = lax.dot_general(
            onehot, emb_hi, (((1,), (0,)), ((), ())),
            preferred_element_type=jnp.float32)
        lo = lax.dot_general(
            onehot, emb_lo, (((1,), (0,)), ((), ())),
            preferred_element_type=jnp.float32)
        out_ref[sl, :] = hi + lo


def kernel(input, embeddings):
    n = input.shape[0] * input.shape[1]
    idx2 = input.reshape(n, 1).astype(jnp.int32)
    embp = jnp.zeros((K_PAD, HEAD_DIM), jnp.float32).at[:NUM_EMB].set(embeddings)
    emb_hi = embp.astype(jnp.bfloat16)
    emb_lo = (embp - emb_hi.astype(jnp.float32)).astype(jnp.bfloat16)
    out = pl.pallas_call(
        _gather_kernel,
        grid=(n // M_BLOCK,),
        in_specs=[
            pl.BlockSpec((M_BLOCK, 1), lambda i: (i, 0)),
            pl.BlockSpec((K_PAD, HEAD_DIM), lambda i: (0, 0)),
            pl.BlockSpec((K_PAD, HEAD_DIM), lambda i: (0, 0)),
        ],
        out_specs=pl.BlockSpec((M_BLOCK, HEAD_DIM), lambda i: (i, 0)),
        out_shape=jax.ShapeDtypeStruct((n, HEAD_DIM), jnp.float32),
    )(idx2, emb_hi, emb_lo)
    return out.reshape(input.shape[0], input.shape[1], HEAD_DIM)


# speedup vs baseline: 1.6393x; 1.6393x over previous
import functools
import jax
import jax.numpy as jnp
from jax import lax
from jax.experimental import pallas as pl
from jax.experimental.pallas import tpu as pltpu
from jax.experimental.pallas import tpu_sc as plsc

HEAD_DIM = 64
NUM_EMB = 257
SEQ = 2048
PAIR_DIM = 2 * HEAD_DIM
B_PAIRS = SEQ * SEQ // 2
NC = 2
NS = 16
NW = NC * NS
B_PER_W = B_PAIRS // NW
IDX_MINOR = 128
CHUNK = 512
ROWS_PER_CHUNK = CHUNK // IDX_MINOR
N_CHUNKS = B_PER_W // CHUNK


def _sc_body(idx_hbm, table_hbm, out_hbm, idx_v, rows_v, sem):
    wid = lax.axis_index("s") * NC + lax.axis_index("c")
    base = wid * B_PER_W

    def body(c, carry):
        off = pl.multiple_of(base + c * CHUNK, CHUNK)
        idx_row = pl.multiple_of(off // IDX_MINOR, ROWS_PER_CHUNK)
        pltpu.sync_copy(idx_hbm.at[pl.ds(idx_row, ROWS_PER_CHUNK)], idx_v)
        copies = []
        for j in range(ROWS_PER_CHUNK):
            copies.append(pltpu.async_copy(
                table_hbm.at[idx_v.at[j]],
                rows_v.at[pl.ds(j * IDX_MINOR, IDX_MINOR)],
                sem))
        for cp in copies:
            cp.wait()
        pltpu.sync_copy(rows_v, out_hbm.at[pl.ds(off, CHUNK)])
        return carry

    lax.fori_loop(0, N_CHUNKS, body, 0)


_sc_gather = functools.partial(
    pl.kernel,
    out_type=jax.ShapeDtypeStruct((B_PAIRS, PAIR_DIM), jnp.float32),
    mesh=plsc.VectorSubcoreMesh(core_axis_name="c", subcore_axis_name="s"),
    scratch_types=[
        pltpu.VMEM((ROWS_PER_CHUNK, IDX_MINOR), jnp.int32),
        pltpu.VMEM((CHUNK, PAIR_DIM), jnp.float32),
        pltpu.SemaphoreType.DMA,
    ],
)(_sc_body)


def kernel(input, embeddings):
    idx = input.reshape(-1).astype(jnp.int32)
    pidx = idx[0::2] * NUM_EMB + idx[1::2]
    pidx2d = pidx.reshape(B_PAIRS // IDX_MINOR, IDX_MINOR)
    ptable = jnp.concatenate(
        [
            jnp.broadcast_to(embeddings[:, None, :],
                             (NUM_EMB, NUM_EMB, HEAD_DIM)),
            jnp.broadcast_to(embeddings[None, :, :],
                             (NUM_EMB, NUM_EMB, HEAD_DIM)),
        ],
        axis=-1,
    ).reshape(NUM_EMB * NUM_EMB, PAIR_DIM)
    out = _sc_gather(pidx2d, ptable)
    return out.reshape(SEQ, SEQ, HEAD_DIM)


# speedup vs baseline: 2.7709x; 1.6902x over previous
import functools
import jax
import jax.numpy as jnp
from jax import lax
from jax.experimental import pallas as pl
from jax.experimental.pallas import tpu as pltpu
from jax.experimental.pallas import tpu_sc as plsc

HEAD_DIM = 64
NUM_EMB = 257
K_PAD = 384
SEQ = 2048
NC = 2
NS = 16
NW = NC * NS
NI_PER_W = SEQ // NW
JCHUNK = 512
NJC = SEQ // JCHUNK
NG = JCHUNK // 16
N_PAIRS = NI_PER_W * NJC // 2


def _sc_body(idx_hbm, table_hbm, out_hbm, table_v, idx_v, m_v, sems):
    wid = lax.axis_index("s") * NC + lax.axis_index("c")
    i0 = wid * NI_PER_W
    pltpu.sync_copy(table_hbm, table_v)

    def make_chunk(half):
        def chunk(k):
            i = i0 + k // NJC
            jc = lax.rem(k, NJC)
            jof = pl.multiple_of(jc * JCHUNK, JCHUNK)

            @pl.when(jc == 0)
            def _():
                pltpu.sync_copy(idx_hbm.at[i], idx_v)

            def g_body(g, carry):
                gg = jc * NG + g
                v = idx_v[gg // 8, pl.ds(lax.rem(gg, 8) * 16, 16)]
                for d in range(HEAD_DIM):
                    dvec = jnp.full((16,), d, jnp.int32)
                    vals = plsc.load_gather(table_v, [dvec, v])
                    m_v[half, d, pl.ds(g * 16, 16)] = vals
                return carry

            lax.fori_loop(0, NG, g_body, 0)
            pltpu.async_copy(
                m_v.at[half],
                out_hbm.at[i, :, pl.ds(jof, JCHUNK)],
                sems.at[half])
        return chunk

    chunk0 = make_chunk(0)
    chunk1 = make_chunk(1)

    def pair_body(p, carry):
        @pl.when(p > 0)
        def _():
            pltpu.make_async_copy(
                m_v.at[0], out_hbm.at[i0, :, pl.ds(0, JCHUNK)],
                sems.at[0]).wait()
        chunk0(2 * p)

        @pl.when(p > 0)
        def _():
            pltpu.make_async_copy(
                m_v.at[1], out_hbm.at[i0, :, pl.ds(0, JCHUNK)],
                sems.at[1]).wait()
        chunk1(2 * p + 1)
        return carry

    lax.fori_loop(0, N_PAIRS, pair_body, 0)
    pltpu.make_async_copy(
        m_v.at[0], out_hbm.at[i0, :, pl.ds(0, JCHUNK)], sems.at[0]).wait()
    pltpu.make_async_copy(
        m_v.at[1], out_hbm.at[i0, :, pl.ds(0, JCHUNK)], sems.at[1]).wait()


_sc_gather = functools.partial(
    pl.kernel,
    out_type=jax.ShapeDtypeStruct((SEQ, HEAD_DIM, SEQ), jnp.float32),
    mesh=plsc.VectorSubcoreMesh(core_axis_name="c", subcore_axis_name="s"),
    scratch_types=[
        pltpu.VMEM((HEAD_DIM, K_PAD), jnp.float32),
        pltpu.VMEM((16, 128), jnp.int32),
        pltpu.VMEM((2, HEAD_DIM, JCHUNK), jnp.float32),
        pltpu.SemaphoreType.DMA((2,)),
    ],
    compiler_params=pltpu.CompilerParams(needs_layout_passes=False),
)(_sc_body)


def kernel(input, embeddings):
    idx3 = input.reshape(SEQ, 16, 128).astype(jnp.int32)
    table_t = jnp.zeros((HEAD_DIM, K_PAD), jnp.float32)
    table_t = table_t.at[:, :NUM_EMB].set(embeddings.T)
    out3 = _sc_gather(idx3, table_t)
    return jnp.transpose(out3, (0, 2, 1))


# speedup vs baseline: 4.4568x; 1.6084x over previous
import functools
import jax
import jax.numpy as jnp
from jax import lax
from jax.experimental import pallas as pl
from jax.experimental.pallas import tpu as pltpu
from jax.experimental.pallas import tpu_sc as plsc

HEAD_DIM = 64
NUM_EMB = 257
K_PAD = 384
SEQ = 2048
NC = 2
NS = 16
NW = NC * NS
NI_PER_W = SEQ // NW
JCHUNK = 512
NJC = SEQ // JCHUNK
NG = JCHUNK // 16
N_PAIRS = NI_PER_W * NJC // 2


def _sc_body(idx_hbm, table_hbm, out_hbm, table_v, idx_v, m_v, sems):
    wid = lax.axis_index("s") * NC + lax.axis_index("c")
    i0 = wid * NI_PER_W
    pltpu.sync_copy(table_hbm, table_v)

    def make_chunk(half):
        def chunk(k):
            i = i0 + k // NJC
            jc = lax.rem(k, NJC)
            jof = pl.multiple_of(jc * JCHUNK, JCHUNK)

            @pl.when(jc == 0)
            def _():
                pltpu.sync_copy(idx_hbm.at[i], idx_v)

            @plsc.parallel_loop(0, NG, unroll=2)
            def g_body(g):
                gg = jc * NG + g
                v = idx_v[gg // 8, pl.ds(lax.rem(gg, 8) * 16, 16)]
                for d in range(HEAD_DIM):
                    dvec = jnp.full((16,), d, jnp.int32)
                    vals = plsc.load_gather(table_v, [dvec, v])
                    m_v[half, d, pl.ds(g * 16, 16)] = vals
            pltpu.async_copy(
                m_v.at[half],
                out_hbm.at[i, :, pl.ds(jof, JCHUNK)],
                sems.at[half])
        return chunk

    chunk0 = make_chunk(0)
    chunk1 = make_chunk(1)

    def pair_body(p, carry):
        @pl.when(p > 0)
        def _():
            pltpu.make_async_copy(
                m_v.at[0], out_hbm.at[i0, :, pl.ds(0, JCHUNK)],
                sems.at[0]).wait()
        chunk0(2 * p)

        @pl.when(p > 0)
        def _():
            pltpu.make_async_copy(
                m_v.at[1], out_hbm.at[i0, :, pl.ds(0, JCHUNK)],
                sems.at[1]).wait()
        chunk1(2 * p + 1)
        return carry

    lax.fori_loop(0, N_PAIRS, pair_body, 0)
    pltpu.make_async_copy(
        m_v.at[0], out_hbm.at[i0, :, pl.ds(0, JCHUNK)], sems.at[0]).wait()
    pltpu.make_async_copy(
        m_v.at[1], out_hbm.at[i0, :, pl.ds(0, JCHUNK)], sems.at[1]).wait()


_sc_gather = functools.partial(
    pl.kernel,
    out_type=jax.ShapeDtypeStruct((SEQ, HEAD_DIM, SEQ), jnp.float32),
    mesh=plsc.VectorSubcoreMesh(core_axis_name="c", subcore_axis_name="s"),
    scratch_types=[
        pltpu.VMEM((HEAD_DIM, K_PAD), jnp.float32),
        pltpu.VMEM((16, 128), jnp.int32),
        pltpu.VMEM((2, HEAD_DIM, JCHUNK), jnp.float32),
        pltpu.SemaphoreType.DMA((2,)),
    ],
    compiler_params=pltpu.CompilerParams(needs_layout_passes=False),
)(_sc_body)


def kernel(input, embeddings):
    idx3 = input.reshape(SEQ, 16, 128).astype(jnp.int32)
    table_t = jnp.zeros((HEAD_DIM, K_PAD), jnp.float32)
    table_t = table_t.at[:, :NUM_EMB].set(embeddings.T)
    out3 = _sc_gather(idx3, table_t)
    return jnp.transpose(out3, (0, 2, 1))


# speedup vs baseline: 7.9254x; 1.7783x over previous
import functools
import jax
import jax.numpy as jnp
from jax import lax
from jax.experimental import pallas as pl
from jax.experimental.pallas import tpu as pltpu
from jax.experimental.pallas import tpu_sc as plsc

HEAD_DIM = 64
NUM_EMB = 257
K_PAD = 384
SEQ = 2048
NC = 2
NS = 16
NW = NC * NS
NI_PER_W = SEQ // NW
JCHUNK = 512
NJC = SEQ // JCHUNK
NG = JCHUNK // 16
N_PAIRS = NI_PER_W * NJC // 2


def _sc_body(idx_hbm, table_hbm, out_hbm, table_v, idx_v, m_v, sems):
    wid = lax.axis_index("s") * NC + lax.axis_index("c")
    i0 = wid * NI_PER_W
    pltpu.sync_copy(table_hbm, table_v)

    def make_chunk(half):
        def chunk(k):
            i = i0 + k // NJC
            jc = lax.rem(k, NJC)
            jof = pl.multiple_of(jc * JCHUNK, JCHUNK)

            @pl.when(jc == 0)
            def _():
                pltpu.sync_copy(idx_hbm.at[i], idx_v)

            @plsc.parallel_loop(0, NG, unroll=4)
            def g_body(g):
                gg = jc * NG + g
                v = idx_v[gg // 8, pl.ds(lax.rem(gg, 8) * 16, 16)]
                for d in range(HEAD_DIM):
                    dvec = jnp.full((16,), d, jnp.int32)
                    vals = plsc.load_gather(table_v, [dvec, v])
                    m_v[half, d, pl.ds(g * 16, 16)] = vals
            pltpu.async_copy(
                m_v.at[half],
                out_hbm.at[i, :, pl.ds(jof, JCHUNK)],
                sems.at[half])
        return chunk

    chunk0 = make_chunk(0)
    chunk1 = make_chunk(1)

    def pair_body(p, carry):
        @pl.when(p > 0)
        def _():
            pltpu.make_async_copy(
                m_v.at[0], out_hbm.at[i0, :, pl.ds(0, JCHUNK)],
                sems.at[0]).wait()
        chunk0(2 * p)

        @pl.when(p > 0)
        def _():
            pltpu.make_async_copy(
                m_v.at[1], out_hbm.at[i0, :, pl.ds(0, JCHUNK)],
                sems.at[1]).wait()
        chunk1(2 * p + 1)
        return carry

    lax.fori_loop(0, N_PAIRS, pair_body, 0)
    pltpu.make_async_copy(
        m_v.at[0], out_hbm.at[i0, :, pl.ds(0, JCHUNK)], sems.at[0]).wait()
    pltpu.make_async_copy(
        m_v.at[1], out_hbm.at[i0, :, pl.ds(0, JCHUNK)], sems.at[1]).wait()


_sc_gather = functools.partial(
    pl.kernel,
    out_type=jax.ShapeDtypeStruct((SEQ, HEAD_DIM, SEQ), jnp.float32),
    mesh=plsc.VectorSubcoreMesh(core_axis_name="c", subcore_axis_name="s"),
    scratch_types=[
        pltpu.VMEM((HEAD_DIM, K_PAD), jnp.float32),
        pltpu.VMEM((16, 128), jnp.int32),
        pltpu.VMEM((2, HEAD_DIM, JCHUNK), jnp.float32),
        pltpu.SemaphoreType.DMA((2,)),
    ],
    compiler_params=pltpu.CompilerParams(needs_layout_passes=False),
)(_sc_body)


def kernel(input, embeddings):
    idx3 = input.reshape(SEQ, 16, 128).astype(jnp.int32)
    table_t = jnp.zeros((HEAD_DIM, K_PAD), jnp.float32)
    table_t = table_t.at[:, :NUM_EMB].set(embeddings.T)
    out3 = _sc_gather(idx3, table_t)
    return jnp.transpose(out3, (0, 2, 1))
